# Initial kernel scaffold; baseline (speedup 1.0000x reference)
import jax
import jax.numpy as jnp
from jax.experimental import pallas as pl


def kernel(x, edge_index, W1, b1, t1, W2, b2, t2, ln_g, ln_b, Wc, bc):
    raise NotImplementedError("write your pallas kernel here")


# speedup vs baseline: 2.0302x; 2.0302x over previous
import functools

import jax
import jax.numpy as jnp
from jax import lax
from jax.experimental import pallas as pl
from jax.experimental.pallas import tpu as pltpu
from jax.experimental.pallas import tpu_sc as plsc

N = 10000
D = 128
E = 320000
EPS = 1e-7

H = 64
NPAD = 10240
ROWS_PER_TILE = NPAD // 16
IDXW = 128
BLK_ROWS = 8
SUB_ROWS = 1
CE = SUB_ROWS * IDXW
BLOCKS = 20
EPT = BLK_ROWS * IDXW * BLOCKS
EPAD = EPT * 16
OB = 64


def _sc_aggr_body(hrows, src2d, dst2d, tvec, out0, out1,
                  acc, srcb, dstb, gbuf, pay, obuf, tv, sem):
    c = lax.axis_index("c")
    s = lax.axis_index("s")

    pltpu.sync_copy(tvec, tv)
    vt = tv[...]
    zero = jnp.zeros((16,), jnp.float32)
    coff = c * H

    def _zloop(e, carry):
        for j in range(8):
            pay[e, pl.ds(j * 16, 16)] = zero
        return carry
    lax.fori_loop(0, CE, _zloop, 0)
    r0 = s * ROWS_PER_TILE
    for z in range(ROWS_PER_TILE // CE):
        pltpu.sync_copy(pay, acc.at[pl.ds(r0 + z * CE, CE)])
    plsc.subcore_barrier()

    rbase = s * (EPT // IDXW)

    def _chunk(k, carry):
        rr = rbase + k * BLK_ROWS
        pltpu.sync_copy(src2d.at[pl.ds(rr, BLK_ROWS)], srcb)
        pltpu.sync_copy(dst2d.at[pl.ds(rr, BLK_ROWS)], dstb)

        for sub in range(BLK_ROWS // SUB_ROWS):
            cps = [pltpu.async_copy(hrows.at[srcb.at[sub * SUB_ROWS + j]],
                                    gbuf.at[pl.ds(j * IDXW, IDXW)], sem)
                   for j in range(SUB_ROWS)]
            for cp in cps:
                cp.wait()

            def _comp(e, carry2):
                for j in range(4):
                    g = gbuf[e, pl.ds(coff + j * 16, 16)]
                    m = jnp.maximum(g, 0.0) + EPS
                    w = jnp.exp(vt * m)
                    pay[e, pl.ds(j * 16, 16)] = w * m
                    pay[e, pl.ds(H + j * 16, 16)] = w
                return carry2
            lax.fori_loop(0, CE, _comp, 0)

            for j in range(SUB_ROWS):
                pltpu.sync_copy(pay.at[pl.ds(j * IDXW, IDXW)],
                                acc.at[dstb.at[sub * SUB_ROWS + j]], add=True)
        return carry
    lax.fori_loop(0, BLOCKS, _chunk, 0)
    plsc.subcore_barrier()

    for b in range(ROWS_PER_TILE // OB):
        rb0 = r0 + b * OB
        rb2 = s * (ROWS_PER_TILE // 2) + b * (OB // 2)
        pltpu.sync_copy(acc.at[pl.ds(rb0, OB)], pay.at[pl.ds(0, OB)])

        def _div(e, carry):
            for half in range(2):
                for j in range(4):
                    num = pay[2 * e + half, pl.ds(j * 16, 16)]
                    den = pay[2 * e + half, pl.ds(H + j * 16, 16)]
                    obuf[e, pl.ds(half * H + j * 16, 16)] = num / (den + 1e-16)
            return carry
        lax.fori_loop(0, OB // 2, _div, 0)

        @pl.when(c == 0)
        def _():
            pltpu.sync_copy(obuf, out0.at[pl.ds(rb2, OB // 2)])

        @pl.when(c == 1)
        def _():
            pltpu.sync_copy(obuf, out1.at[pl.ds(rb2, OB // 2)])


_sc_aggr = functools.partial(
    pl.kernel,
    out_type=[jax.ShapeDtypeStruct((NPAD // 2, 2 * H), jnp.float32),
              jax.ShapeDtypeStruct((NPAD // 2, 2 * H), jnp.float32)],
    mesh=plsc.VectorSubcoreMesh(core_axis_name="c", subcore_axis_name="s"),
    scratch_types=[
        pltpu.VMEM_SHARED((NPAD, 2 * H), jnp.float32),
        pltpu.VMEM((BLK_ROWS, IDXW), jnp.int32),
        pltpu.VMEM((BLK_ROWS, IDXW), jnp.int32),
        pltpu.VMEM((CE, D), jnp.float32),
        pltpu.VMEM((CE, 2 * H), jnp.float32),
        pltpu.VMEM((OB // 2, 2 * H), jnp.float32),
        pltpu.VMEM((16,), jnp.float32),
        pltpu.SemaphoreType.DMA,
    ],
)(_sc_aggr_body)


BR = 1000


def _t1_body(x_ref, a0_ref, a1_ref, w_ref, b_ref, o_ref):
    xa = x_ref[...] + jnp.concatenate([a0_ref[...], a1_ref[...]], axis=1)
    o_ref[...] = lax.dot_general(
        xa, w_ref[...], (((1,), (1,)), ((), ())),
        preferred_element_type=jnp.float32) + b_ref[...]


def _tc_linear(x1, a0, a1, W, b):
    return pl.pallas_call(
        _t1_body,
        grid=(N // BR,),
        in_specs=[
            pl.BlockSpec((BR, D), lambda i: (i, 0)),
            pl.BlockSpec((BR, H), lambda i: (i, 0)),
            pl.BlockSpec((BR, H), lambda i: (i, 0)),
            pl.BlockSpec((D, D), lambda i: (0, 0)),
            pl.BlockSpec((1, D), lambda i: (0, 0)),
        ],
        out_specs=pl.BlockSpec((BR, D), lambda i: (i, 0)),
        out_shape=jax.ShapeDtypeStruct((N, D), jnp.float32),
    )(x1, a0, a1, W, b)


def _t2_body(x1_ref, a0_ref, a1_ref, w_ref, b_ref, g_ref, bl_ref,
             wc_ref, bc_ref, xr_ref, o_ref, s_acc):
    i = pl.program_id(0)

    @pl.when(i == 0)
    def _():
        s_acc[...] = jnp.zeros_like(s_acc)

    x1 = x1_ref[...]
    xa = x1 + jnp.concatenate([a0_ref[...], a1_ref[...]], axis=1)
    h = lax.dot_general(xa, w_ref[...], (((1,), (1,)), ((), ())),
                        preferred_element_type=jnp.float32) + b_ref[...]
    mu = jnp.mean(h, axis=-1, keepdims=True)
    var = jnp.mean((h - mu) ** 2, axis=-1, keepdims=True)
    hn = g_ref[...] * (h - mu) / jnp.sqrt(var + 1e-5) + bl_ref[...]
    x2 = x1 + jnp.maximum(hn, 0.0)
    s_acc[:, 0:D] += jnp.sum(x1, axis=0, keepdims=True)
    s_acc[:, D:2 * D] += jnp.sum(x2, axis=0, keepdims=True)

    @pl.when(i == N // BR - 1)
    def _():
        mean_cat = s_acc[...] * (1.0 / N)
        delta = lax.dot_general(mean_cat, wc_ref[...], (((1,), (1,)), ((), ())),
                                preferred_element_type=jnp.float32) + bc_ref[...]
        o_ref[...] = xr_ref[...] + delta


def _tc_layer2_head(x1, a0, a1, W, b, g, bl, Wc, bc, xrow):
    return pl.pallas_call(
        _t2_body,
        grid=(N // BR,),
        in_specs=[
            pl.BlockSpec((BR, D), lambda i: (i, 0)),
            pl.BlockSpec((BR, H), lambda i: (i, 0)),
            pl.BlockSpec((BR, H), lambda i: (i, 0)),
            pl.BlockSpec((D, D), lambda i: (0, 0)),
            pl.BlockSpec((1, D), lambda i: (0, 0)),
            pl.BlockSpec((1, D), lambda i: (0, 0)),
            pl.BlockSpec((1, D), lambda i: (0, 0)),
            pl.BlockSpec((D, 2 * D), lambda i: (0, 0)),
            pl.BlockSpec((1, D), lambda i: (0, 0)),
            pl.BlockSpec((1, D), lambda i: (0, 0)),
        ],
        out_specs=pl.BlockSpec((1, D), lambda i: (0, 0)),
        out_shape=jax.ShapeDtypeStruct((1, D), jnp.float32),
        scratch_shapes=[pltpu.VMEM((1, 2 * D), jnp.float32)],
    )(x1, a0, a1, W, b, g, bl, Wc, bc, xrow)


def kernel(x, edge_index, W1, b1, t1, W2, b2, t2, ln_g, ln_b, Wc, bc):
    nodes = x[0]
    src = edge_index[0]
    dst = edge_index[1]
    pad = EPAD - E
    srcp = jnp.concatenate([src, jnp.zeros((pad,), jnp.int32)]).reshape(EPAD // IDXW, IDXW)
    dstp = jnp.concatenate([dst, jnp.full((pad,), NPAD - 1, jnp.int32)]).reshape(EPAD // IDXW, IDXW)

    b1r = b1.reshape(1, D)
    b2r = b2.reshape(1, D)
    gr = ln_g.reshape(1, D)
    blr = ln_b.reshape(1, D)
    bcr = bc.reshape(1, D)
    t1v = jnp.full((16,), 1.0, jnp.float32) * t1
    t2v = jnp.full((16,), 1.0, jnp.float32) * t2

    a1a, a1b = _sc_aggr(nodes, srcp, dstp, t1v)
    a1a = a1a.reshape(NPAD, H)
    a1b = a1b.reshape(NPAD, H)
    x1 = _tc_linear(nodes, a1a, a1b, W1, b1r)
    a2a, a2b = _sc_aggr(x1, srcp, dstp, t2v)
    a2a = a2a.reshape(NPAD, H)
    a2b = a2b.reshape(NPAD, H)
    row = _tc_layer2_head(x1, a2a, a2b, W2, b2r, gr, blr, Wc, bcr, nodes[0:1])
    return x.at[:, 0, :].set(row)


# speedup vs baseline: 4.3778x; 2.1563x over previous
import functools

import jax
import jax.numpy as jnp
from jax import lax
from jax.experimental import pallas as pl
from jax.experimental.pallas import tpu as pltpu
from jax.experimental.pallas import tpu_sc as plsc

N = 10000
D = 128
E = 320000
EPS = 1e-7

H = 64
NPAD = 10240
ROWS_PER_TILE = NPAD // 16
IDXW = 128
BLK_ROWS = 8
SUB_ROWS = 1
CE = SUB_ROWS * IDXW
BLOCKS = 20
EPT = BLK_ROWS * IDXW * BLOCKS
EPAD = EPT * 16
OB = 64


def _sc_aggr_body(hrows, src2d, dst2d, tvec, out0, out1,
                  acc, srcb, dstb, gbuf, pay, obuf, tv, sem):
    c = lax.axis_index("c")
    s = lax.axis_index("s")

    pltpu.sync_copy(tvec, tv)
    vt = tv[...]
    zero = jnp.zeros((16,), jnp.float32)
    coff = c * H

    @plsc.parallel_loop(0, CE, 1, unroll=4)
    def _zloop(e):
        for j in range(8):
            pay[e, pl.ds(j * 16, 16)] = zero
    r0 = s * ROWS_PER_TILE
    for z in range(ROWS_PER_TILE // CE):
        pltpu.sync_copy(pay, acc.at[pl.ds(r0 + z * CE, CE)])
    plsc.subcore_barrier()

    rbase = s * (EPT // IDXW)

    def _chunk(k, carry):
        rr = rbase + k * BLK_ROWS
        pltpu.sync_copy(src2d.at[pl.ds(rr, BLK_ROWS)], srcb)
        pltpu.sync_copy(dst2d.at[pl.ds(rr, BLK_ROWS)], dstb)
        pltpu.async_copy(hrows.at[srcb.at[0]], gbuf, sem)

        for sub in range(BLK_ROWS):
            pltpu.make_async_copy(hrows.at[srcb.at[sub]], gbuf, sem).wait()

            @plsc.parallel_loop(0, CE, 1, unroll=2)
            def _comp(e):
                for j in range(4):
                    g = gbuf[e, pl.ds(coff + j * 16, 16)]
                    m = jnp.maximum(g, 0.0) + EPS
                    w = jnp.exp(vt * m)
                    pay[e, pl.ds(j * 16, 16)] = w * m
                    pay[e, pl.ds(H + j * 16, 16)] = w

            if sub < BLK_ROWS - 1:
                pltpu.async_copy(hrows.at[srcb.at[sub + 1]], gbuf, sem)
            pltpu.sync_copy(pay, acc.at[dstb.at[sub]], add=True)
        return carry
    lax.fori_loop(0, BLOCKS, _chunk, 0)
    plsc.subcore_barrier()

    for b in range(ROWS_PER_TILE // OB):
        rb0 = r0 + b * OB
        rb2 = s * (ROWS_PER_TILE // 2) + b * (OB // 2)
        pltpu.sync_copy(acc.at[pl.ds(rb0, OB)], pay.at[pl.ds(0, OB)])

        @plsc.parallel_loop(0, OB // 2, 1, unroll=2)
        def _div(e):
            for half in range(2):
                for j in range(4):
                    num = pay[2 * e + half, pl.ds(j * 16, 16)]
                    den = pay[2 * e + half, pl.ds(H + j * 16, 16)]
                    obuf[e, pl.ds(half * H + j * 16, 16)] = num / (den + 1e-16)

        @pl.when(c == 0)
        def _():
            pltpu.sync_copy(obuf, out0.at[pl.ds(rb2, OB // 2)])

        @pl.when(c == 1)
        def _():
            pltpu.sync_copy(obuf, out1.at[pl.ds(rb2, OB // 2)])


_sc_aggr = functools.partial(
    pl.kernel,
    out_type=[jax.ShapeDtypeStruct((NPAD // 2, 2 * H), jnp.float32),
              jax.ShapeDtypeStruct((NPAD // 2, 2 * H), jnp.float32)],
    mesh=plsc.VectorSubcoreMesh(core_axis_name="c", subcore_axis_name="s"),
    scratch_types=[
        pltpu.VMEM_SHARED((NPAD, 2 * H), jnp.float32),
        pltpu.VMEM((BLK_ROWS, IDXW), jnp.int32),
        pltpu.VMEM((BLK_ROWS, IDXW), jnp.int32),
        pltpu.VMEM((CE, D), jnp.float32),
        pltpu.VMEM((CE, 2 * H), jnp.float32),
        pltpu.VMEM((OB // 2, 2 * H), jnp.float32),
        pltpu.VMEM((16,), jnp.float32),
        pltpu.SemaphoreType.DMA,
    ],
)(_sc_aggr_body)


BR = 1000


def _t1_body(x_ref, a0_ref, a1_ref, w_ref, b_ref, o_ref):
    xa = x_ref[...] + jnp.concatenate([a0_ref[...], a1_ref[...]], axis=1)
    o_ref[...] = lax.dot_general(
        xa, w_ref[...], (((1,), (1,)), ((), ())),
        preferred_element_type=jnp.float32) + b_ref[...]


def _tc_linear(x1, a0, a1, W, b):
    return pl.pallas_call(
        _t1_body,
        grid=(N // BR,),
        in_specs=[
            pl.BlockSpec((BR, D), lambda i: (i, 0)),
            pl.BlockSpec((BR, H), lambda i: (i, 0)),
            pl.BlockSpec((BR, H), lambda i: (i, 0)),
            pl.BlockSpec((D, D), lambda i: (0, 0)),
            pl.BlockSpec((1, D), lambda i: (0, 0)),
        ],
        out_specs=pl.BlockSpec((BR, D), lambda i: (i, 0)),
        out_shape=jax.ShapeDtypeStruct((N, D), jnp.float32),
    )(x1, a0, a1, W, b)


def _t2_body(x1_ref, a0_ref, a1_ref, w_ref, b_ref, g_ref, bl_ref,
             wc_ref, bc_ref, xr_ref, o_ref, s_acc):
    i = pl.program_id(0)

    @pl.when(i == 0)
    def _():
        s_acc[...] = jnp.zeros_like(s_acc)

    x1 = x1_ref[...]
    xa = x1 + jnp.concatenate([a0_ref[...], a1_ref[...]], axis=1)
    h = lax.dot_general(xa, w_ref[...], (((1,), (1,)), ((), ())),
                        preferred_element_type=jnp.float32) + b_ref[...]
    mu = jnp.mean(h, axis=-1, keepdims=True)
    var = jnp.mean((h - mu) ** 2, axis=-1, keepdims=True)
    hn = g_ref[...] * (h - mu) / jnp.sqrt(var + 1e-5) + bl_ref[...]
    x2 = x1 + jnp.maximum(hn, 0.0)
    s_acc[:, 0:D] += jnp.sum(x1, axis=0, keepdims=True)
    s_acc[:, D:2 * D] += jnp.sum(x2, axis=0, keepdims=True)

    @pl.when(i == N // BR - 1)
    def _():
        mean_cat = s_acc[...] * (1.0 / N)
        delta = lax.dot_general(mean_cat, wc_ref[...], (((1,), (1,)), ((), ())),
                                preferred_element_type=jnp.float32) + bc_ref[...]
        o_ref[...] = xr_ref[...] + delta


def _tc_layer2_head(x1, a0, a1, W, b, g, bl, Wc, bc, xrow):
    return pl.pallas_call(
        _t2_body,
        grid=(N // BR,),
        in_specs=[
            pl.BlockSpec((BR, D), lambda i: (i, 0)),
            pl.BlockSpec((BR, H), lambda i: (i, 0)),
            pl.BlockSpec((BR, H), lambda i: (i, 0)),
            pl.BlockSpec((D, D), lambda i: (0, 0)),
            pl.BlockSpec((1, D), lambda i: (0, 0)),
            pl.BlockSpec((1, D), lambda i: (0, 0)),
            pl.BlockSpec((1, D), lambda i: (0, 0)),
            pl.BlockSpec((D, 2 * D), lambda i: (0, 0)),
            pl.BlockSpec((1, D), lambda i: (0, 0)),
            pl.BlockSpec((1, D), lambda i: (0, 0)),
        ],
        out_specs=pl.BlockSpec((1, D), lambda i: (0, 0)),
        out_shape=jax.ShapeDtypeStruct((1, D), jnp.float32),
        scratch_shapes=[pltpu.VMEM((1, 2 * D), jnp.float32)],
    )(x1, a0, a1, W, b, g, bl, Wc, bc, xrow)


def kernel(x, edge_index, W1, b1, t1, W2, b2, t2, ln_g, ln_b, Wc, bc):
    nodes = x[0]
    src = edge_index[0]
    dst = edge_index[1]
    pad = EPAD - E
    srcp = jnp.concatenate([src, jnp.zeros((pad,), jnp.int32)]).reshape(EPAD // IDXW, IDXW)
    dstp = jnp.concatenate([dst, jnp.full((pad,), NPAD - 1, jnp.int32)]).reshape(EPAD // IDXW, IDXW)

    b1r = b1.reshape(1, D)
    b2r = b2.reshape(1, D)
    gr = ln_g.reshape(1, D)
    blr = ln_b.reshape(1, D)
    bcr = bc.reshape(1, D)
    t1v = jnp.full((16,), 1.0, jnp.float32) * t1
    t2v = jnp.full((16,), 1.0, jnp.float32) * t2

    a1a, a1b = _sc_aggr(nodes, srcp, dstp, t1v)
    a1a = a1a.reshape(NPAD, H)
    a1b = a1b.reshape(NPAD, H)
    x1 = _tc_linear(nodes, a1a, a1b, W1, b1r)
    a2a, a2b = _sc_aggr(x1, srcp, dstp, t2v)
    a2a = a2a.reshape(NPAD, H)
    a2b = a2b.reshape(NPAD, H)
    row = _tc_layer2_head(x1, a2a, a2b, W2, b2r, gr, blr, Wc, bcr, nodes[0:1])
    return x.at[:, 0, :].set(row)
